# baseline (speedup 1.0000x reference)
import functools

import jax
import jax.numpy as jnp
from jax import lax
from jax.experimental import pallas as pl
from jax.experimental.pallas import tpu as pltpu
from jax.experimental.pallas import tpu_sc as plsc

VOCAB = 100001
EMBED_DIM = 32
BATCH = 16384

_Q = 4
_QROWS = (VOCAB + _Q - 1) // _Q
_LINE = _Q * EMBED_DIM

_info = plsc.get_sparse_core_info()
_NC = _info.num_cores
_NS = _info.num_subcores
_NW = _NC * _NS
_N = BATCH // _NW
_NBLK = _N // 16

_mesh = plsc.VectorSubcoreMesh(core_axis_name="c", subcore_axis_name="s")


@functools.partial(
    pl.kernel,
    mesh=_mesh,
    out_type=jax.ShapeDtypeStruct((EMBED_DIM, BATCH), jnp.float32),
    scratch_types=[
        pltpu.VMEM((_N,), jnp.int32),
        pltpu.VMEM((_N,), jnp.int32),
        pltpu.VMEM((_N, _LINE), jnp.float32),
        pltpu.VMEM((EMBED_DIM, _N), jnp.float32),
        pltpu.SemaphoreType.DMA,
    ],
    compiler_params=pltpu.CompilerParams(
        use_tc_tiling_on_sc=True, needs_layout_passes=False
    ),
)
def _gather_kernel(idx_hbm, t4_hbm, out_hbm, idx_v, q_v, rows_v, out_b, sem):
    wid = lax.axis_index("s") * _NC + lax.axis_index("c")
    base = wid * _N
    pltpu.sync_copy(idx_hbm.at[pl.ds(base, _N)], idx_v)

    def _quarter(i, carry):
        v = idx_v[pl.ds(i * 16, 16)]
        q_v[pl.ds(i * 16, 16)] = v >> 2
        return carry

    lax.fori_loop(0, _NBLK, _quarter, 0)
    pltpu.async_copy(t4_hbm.at[q_v], rows_v, sem).wait()

    lane = lax.broadcasted_iota(jnp.int32, (16,), 0)

    def _select(i, carry):
        row_idx = i * 16 + lane
        cbase = (idx_v[pl.ds(i * 16, 16)] & 3) * EMBED_DIM
        for d in range(EMBED_DIM):
            out_b[d, pl.ds(i * 16, 16)] = plsc.load_gather(
                rows_v, [row_idx, cbase + d]
            )
        return carry

    lax.fori_loop(0, _NBLK, _select, 0)
    pltpu.sync_copy(out_b, out_hbm.at[:, pl.ds(base, _N)])


def kernel(inputs, table):
    t4 = jnp.pad(table, ((0, _QROWS * _Q - VOCAB), (0, 0))).reshape(_QROWS, _LINE)
    out_t = _gather_kernel(inputs.astype(jnp.int32), t4)
    return out_t.T


# speedup vs baseline: 1.4048x; 1.4048x over previous
import functools

import jax
import jax.numpy as jnp
from jax import lax
from jax.experimental import pallas as pl
from jax.experimental.pallas import tpu as pltpu
from jax.experimental.pallas import tpu_sc as plsc

VOCAB = 100001
EMBED_DIM = 32
BATCH = 16384

_info = plsc.get_sparse_core_info()
_NC = _info.num_cores
_NS = _info.num_subcores
_NW = _NC * _NS
_N = BATCH // _NW

_R = EMBED_DIM // 8
_C = BATCH // 128
_CW = _C // _NW

_mesh = plsc.VectorSubcoreMesh(core_axis_name="c", subcore_axis_name="s")


@functools.partial(
    pl.kernel,
    mesh=_mesh,
    out_type=jax.ShapeDtypeStruct((_R, _C, 8, 128), jnp.float32),
    scratch_types=[
        pltpu.VMEM((_N,), jnp.int32),
        pltpu.VMEM((_N, EMBED_DIM), jnp.float32),
        pltpu.VMEM((_R, _CW, 8, 128), jnp.float32),
        pltpu.SemaphoreType.DMA,
    ],
    compiler_params=pltpu.CompilerParams(
        use_tc_tiling_on_sc=False, needs_layout_passes=False
    ),
)
def _gather_kernel(idx_hbm, table_hbm, out_hbm, idx_v, rows_v, x_l, sem):
    wid = lax.axis_index("s") * _NC + lax.axis_index("c")
    base = wid * _N
    pltpu.sync_copy(idx_hbm.at[pl.ds(base, _N)], idx_v)
    pltpu.async_copy(table_hbm.at[idx_v], rows_v, sem).wait()

    lane = lax.broadcasted_iota(jnp.int32, (16,), 0)

    def _rearrange(cc, carry):
        for r in range(_R):
            for sr in range(8):
                col = jnp.full((16,), 8 * r + sr, jnp.int32)
                for lb in range(8):
                    row_idx = cc * 128 + lb * 16 + lane
                    x_l[r, cc, sr, pl.ds(lb * 16, 16)] = plsc.load_gather(
                        rows_v, [row_idx, col]
                    )
        return carry

    lax.fori_loop(0, _CW, _rearrange, 0)
    pltpu.sync_copy(x_l, out_hbm.at[:, pl.ds(wid * _CW, _CW)])


def kernel(inputs, table):
    x = _gather_kernel(inputs.astype(jnp.int32), table)
    return jnp.transpose(x, (1, 3, 0, 2)).reshape(BATCH, EMBED_DIM)
